# Initial kernel scaffold; baseline (speedup 1.0000x reference)
import jax
import jax.numpy as jnp
from jax.experimental import pallas as pl


def kernel(operations, items, related_items, materials, resources, need_for_resources_edge_index, need_for_materials_edge_index, precedences_edge_index, params):
    raise NotImplementedError("write your pallas kernel here")


# speedup vs baseline: 7.1537x; 7.1537x over previous
import functools

import jax
import jax.numpy as jnp
from jax import lax
from jax.experimental import pallas as pl
from jax.experimental.pallas import tpu as pltpu
from jax.experimental.pallas import tpu_sc as plsc

N = 10000
E = 320000
NS = 16
NC = 2
CH = 128
EP = E // NS
NCHF = EP // CH
TR = EP - NCHF * CH
NACC = 10240
GARB = N
ZR = NACC // NS
NI = 10240
IT_CH = 64
IT_NCH = (NI // 32) // IT_CH


def _edge_loop(table, accum, e_src, e_dst, ebase, sidx, didx, rows2,
               sem_g, sem_i):
    def load_idx(j, b):
        pltpu.async_copy(e_src.at[pl.ds(ebase + j * CH, CH)], sidx.at[b],
                         sem_i)
        pltpu.async_copy(e_dst.at[pl.ds(ebase + j * CH, CH)], didx.at[b],
                         sem_i)

    def wait_idx(b):
        pltpu.make_async_copy(e_src.at[pl.ds(0, CH)], sidx.at[b],
                              sem_i).wait()
        pltpu.make_async_copy(e_dst.at[pl.ds(0, CH)], didx.at[b],
                              sem_i).wait()

    def gather(b):
        pltpu.async_copy(table.at[sidx.at[b]], rows2.at[b], sem_g)

    def wait_gather(b):
        pltpu.make_async_copy(table.at[sidx.at[0]], rows2.at[b],
                              sem_g).wait()

    def scatter(b):
        pltpu.sync_copy(rows2.at[b], accum.at[didx.at[b]], add=True)

    load_idx(0, 0)
    wait_idx(0)
    gather(0)
    load_idx(1, 1)

    def body(g, carry):
        for b in (0, 1):
            j = 2 * g + b
            wait_gather(b)

            @pl.when(j + 1 < NCHF)
            def _():
                wait_idx(1 - b)
                gather(1 - b)

            @pl.when(j + 2 < NCHF)
            def _():
                load_idx(j + 2, b)

            scatter(b)
        return carry

    lax.fori_loop(0, NCHF // 2, body, 0)

    pltpu.sync_copy(e_src.at[pl.ds(ebase + NCHF * CH, TR)],
                    sidx.at[0, pl.ds(0, TR)])
    pltpu.sync_copy(e_dst.at[pl.ds(ebase + NCHF * CH, TR)],
                    didx.at[0, pl.ds(0, TR)])
    pad_s = lax.iota(jnp.int32, 16)
    pad_d = GARB + pad_s
    for t in range(TR, CH, 16):
        sidx[0, pl.ds(t, 16)] = pad_s
        didx[0, pl.ds(t, 16)] = pad_d
    pltpu.async_copy(table.at[sidx.at[0]], rows2.at[0], sem_g)
    pltpu.make_async_copy(table.at[sidx.at[0]], rows2.at[0], sem_g).wait()
    pltpu.sync_copy(rows2.at[0], accum.at[didx.at[0]], add=True)


@functools.cache
def _get_sc_kernel():
  mesh = plsc.VectorSubcoreMesh(
      core_axis_name="c", subcore_axis_name="s", num_cores=NC, num_subcores=NS)

  @functools.partial(
    pl.kernel,
    out_type=[
        jax.ShapeDtypeStruct((NI, 128), jnp.float32),
        jax.ShapeDtypeStruct((NACC, 128), jnp.float32),
        jax.ShapeDtypeStruct((NACC, 128), jnp.float32),
        jax.ShapeDtypeStruct((NACC, 128), jnp.float32),
        jax.ShapeDtypeStruct((NACC, 128), jnp.float32),
    ],
    mesh=mesh,
    scratch_types=[
        pltpu.VMEM((2, CH), jnp.int32),
        pltpu.VMEM((2, CH), jnp.int32),
        pltpu.VMEM((2, CH, 128), jnp.float32),
        pltpu.VMEM((IT_NCH, IT_CH), jnp.int32),
        pltpu.VMEM_SHARED((NACC, 128), jnp.float32),
        pltpu.SemaphoreType.DMA,
        pltpu.SemaphoreType.DMA,
    ],
  )
  def _sc_gather_agg(ops_hbm, items_hbm, rm_hbm, prec0, prec1, rese0, rese1,
                     mate0, mate1, rel, z128,
                     out_items, out_preds, out_succs, out_res, out_mat,
                     sidx, didx, rows128, ridx, acc, sem_g, sem_i):
      cid = lax.axis_index("c")
      sid = lax.axis_index("s")
      wid = cid * NS + sid
      ebase = sid * EP

      def zero_stripe():
          pltpu.sync_copy(z128.at[pl.ds(0, ZR)], acc.at[pl.ds(sid * ZR, ZR)])

      def flush(out0, out1):
          @pl.when(cid == 0)
          def _():
              pltpu.sync_copy(acc.at[pl.ds(sid * ZR, ZR)],
                              out0.at[pl.ds(sid * ZR, ZR)])

          @pl.when(cid == 1)
          def _():
              pltpu.sync_copy(acc.at[pl.ds(sid * ZR, ZR)],
                              out1.at[pl.ds(sid * ZR, ZR)])

      zero_stripe()

      pltpu.sync_copy(rel.at[wid], ridx)
      for c in range(IT_NCH):
          pltpu.async_copy(items_hbm.at[ridx.at[c]],
                           rows128.at[0, pl.ds(0, IT_CH)], sem_g)
          pltpu.make_async_copy(items_hbm.at[ridx.at[c]],
                                rows128.at[0, pl.ds(0, IT_CH)], sem_g).wait()
          pltpu.sync_copy(
              rows128.at[0, pl.ds(0, IT_CH)],
              out_items.at[pl.ds(wid * IT_NCH * IT_CH + c * IT_CH, IT_CH)])

      plsc.subcore_barrier()

      @pl.when(cid == 0)
      def _():
          _edge_loop(ops_hbm, acc, prec1, prec0, ebase, sidx, didx, rows128,
                     sem_g, sem_i)

      @pl.when(cid == 1)
      def _():
          _edge_loop(ops_hbm, acc, prec0, prec1, ebase, sidx, didx, rows128,
                     sem_g, sem_i)

      plsc.subcore_barrier()
      flush(out_preds, out_succs)
      plsc.subcore_barrier()
      zero_stripe()
      plsc.subcore_barrier()

      @pl.when(cid == 0)
      def _():
          _edge_loop(rm_hbm, acc, rese1, rese0, ebase, sidx, didx, rows128,
                     sem_g, sem_i)

      @pl.when(cid == 1)
      def _():
          _edge_loop(rm_hbm, acc, mate1, mate0, ebase, sidx, didx, rows128,
                     sem_g, sem_i)

      plsc.subcore_barrier()
      flush(out_res, out_mat)

  return _sc_gather_agg


def _elu(x):
    return jnp.where(x > 0, x, jnp.exp(x) - 1.0)


def _tc_body(ops_ref, itm_ref, agp_ref, ags_ref, agr_ref, agm_ref, *refs):
    prefs, out_ref = refs[:-1], refs[-1]

    def layers(i):
        base = i * 6
        return [(prefs[base + 2 * k], prefs[base + 2 * k + 1])
                for k in range(3)]

    def mlp(x, ws):
        for k, (w, b) in enumerate(ws):
            x = jnp.dot(x, w[...], preferred_element_type=jnp.float32) + b[...]
            if k < 2:
                x = _elu(x)
        return x

    p = mlp(agp_ref[...], layers(0))
    s = mlp(ags_ref[...], layers(1))
    r = mlp(agr_ref[...], layers(2))
    m = mlp(agm_ref[...], layers(3))
    it = mlp(itm_ref[...], layers(4))
    se = mlp(ops_ref[...], layers(5))
    comb = jnp.concatenate([p, s, r, m, it, se], axis=-1)
    out_ref[...] = mlp(comb, layers(6))


_BROW = 400


def _tc_mlps(ops, itm, agp, ags, agr, agm, flat_params):
    row_in = [
        pl.BlockSpec((_BROW, a.shape[1]), lambda i: (i, 0))
        for a in (ops, itm, agp, ags, agr, agm)
    ]
    w_in = [pl.BlockSpec(w.shape, lambda i: (0,) * w.ndim) for w in flat_params]
    return pl.pallas_call(
        _tc_body,
        grid=(N // _BROW,),
        in_specs=row_in + w_in,
        out_specs=pl.BlockSpec((_BROW, 128), lambda i: (i, 0)),
        out_shape=jax.ShapeDtypeStruct((N, 128), jnp.float32),
        compiler_params=pltpu.CompilerParams(
            dimension_semantics=("arbitrary",)),
    )(ops, itm, agp, ags, agr, agm, *flat_params)


def kernel(operations, items, related_items, materials, resources,
           need_for_resources_edge_index, need_for_materials_edge_index,
           precedences_edge_index, params):
    i32 = jnp.int32
    prec = precedences_edge_index.astype(i32)
    rese = need_for_resources_edge_index.astype(i32)
    mate = need_for_materials_edge_index.astype(i32)

    rel = jnp.concatenate(
        [related_items.astype(i32),
         jnp.zeros((NI - N,), i32)]).reshape(32, IT_NCH, IT_CH)
    z128 = jnp.zeros((ZR, 128), jnp.float32)
    rm = jnp.concatenate([resources, materials], axis=1)

    it_rows, agp, ags, agr, agm = _get_sc_kernel()(
        operations, items, rm, prec[0], prec[1], rese[0], rese[1],
        mate[0], mate[1], rel, z128)

    flat_params = []
    for name in ("predecessors", "successors", "resources", "materials",
                 "items", "self", "combined"):
        for layer in params[name]:
            flat_params.append(layer["W"])
            flat_params.append(layer["b"].reshape(1, -1))

    return _tc_mlps(operations, it_rows[:N], agp[:N], ags[:N],
                    agr[:N, :64], agm[:N, 64:], flat_params)


# speedup vs baseline: 7.3635x; 1.0293x over previous
import functools

import jax
import jax.numpy as jnp
from jax import lax
from jax.experimental import pallas as pl
from jax.experimental.pallas import tpu as pltpu
from jax.experimental.pallas import tpu_sc as plsc

N = 10000
E = 320000
NS = 16
NC = 2
CH = 128
EP = E // NS
NCHF = EP // CH
TR = EP - NCHF * CH
NACC = 10240
GARB = N
ZR = NACC // NS
NI = 10240
IT_CH = 64
IT_NCH = (NI // 32) // IT_CH


def _edge_loop(table, accum, e_src, e_dst, ebase, sidx, didx, rows2,
               sem_g, sem_i):
    def load_idx(j, b):
        pltpu.async_copy(e_src.at[pl.ds(ebase + j * CH, CH)], sidx.at[b],
                         sem_i)
        pltpu.async_copy(e_dst.at[pl.ds(ebase + j * CH, CH)], didx.at[b],
                         sem_i)

    def wait_idx(b):
        pltpu.make_async_copy(e_src.at[pl.ds(0, CH)], sidx.at[b],
                              sem_i).wait()
        pltpu.make_async_copy(e_dst.at[pl.ds(0, CH)], didx.at[b],
                              sem_i).wait()

    def gather(b):
        pltpu.async_copy(table.at[sidx.at[b]], rows2.at[b], sem_g)

    def wait_gather(b):
        pltpu.make_async_copy(table.at[sidx.at[0]], rows2.at[b],
                              sem_g).wait()

    def scatter(b):
        pltpu.sync_copy(rows2.at[b], accum.at[didx.at[b]], add=True)

    load_idx(0, 0)
    wait_idx(0)
    gather(0)
    load_idx(1, 1)

    def body(g, carry):
        for b in (0, 1):
            j = 2 * g + b
            wait_gather(b)

            @pl.when(j + 1 < NCHF)
            def _():
                wait_idx(1 - b)
                gather(1 - b)

            @pl.when(j + 2 < NCHF)
            def _():
                load_idx(j + 2, b)

            scatter(b)
        return carry

    lax.fori_loop(0, NCHF // 2, body, 0)

    pltpu.sync_copy(e_src.at[pl.ds(ebase + NCHF * CH, TR)],
                    sidx.at[0, pl.ds(0, TR)])
    pltpu.sync_copy(e_dst.at[pl.ds(ebase + NCHF * CH, TR)],
                    didx.at[0, pl.ds(0, TR)])
    pad_s = lax.iota(jnp.int32, 16)
    pad_d = GARB + pad_s
    for t in range(TR, CH, 16):
        sidx[0, pl.ds(t, 16)] = pad_s
        didx[0, pl.ds(t, 16)] = pad_d
    pltpu.async_copy(table.at[sidx.at[0]], rows2.at[0], sem_g)
    pltpu.make_async_copy(table.at[sidx.at[0]], rows2.at[0], sem_g).wait()
    pltpu.sync_copy(rows2.at[0], accum.at[didx.at[0]], add=True)


@functools.cache
def _get_sc_kernel():
  mesh = plsc.VectorSubcoreMesh(
      core_axis_name="c", subcore_axis_name="s", num_cores=NC, num_subcores=NS)

  @functools.partial(
    pl.kernel,
    out_type=[
        jax.ShapeDtypeStruct((NI, 128), jnp.float32),
        jax.ShapeDtypeStruct((NACC, 128), jnp.float32),
        jax.ShapeDtypeStruct((NACC, 128), jnp.float32),
        jax.ShapeDtypeStruct((NACC, 128), jnp.float32),
        jax.ShapeDtypeStruct((NACC, 128), jnp.float32),
    ],
    mesh=mesh,
    scratch_types=[
        pltpu.VMEM((2, CH), jnp.int32),
        pltpu.VMEM((2, CH), jnp.int32),
        pltpu.VMEM((2, CH, 128), jnp.float32),
        pltpu.VMEM((IT_NCH, IT_CH), jnp.int32),
        pltpu.VMEM_SHARED((NACC, 128), jnp.float32),
        pltpu.SemaphoreType.DMA,
        pltpu.SemaphoreType.DMA,
    ],
  )
  def _sc_gather_agg(ops_hbm, items_hbm, rm_hbm, prec0, prec1, rese0, rese1,
                     mate0, mate1, rel, z128,
                     out_items, out_preds, out_succs, out_res, out_mat,
                     sidx, didx, rows128, ridx, acc, sem_g, sem_i):
      cid = lax.axis_index("c")
      sid = lax.axis_index("s")
      wid = cid * NS + sid
      ebase = sid * EP

      def zero_stripe():
          pltpu.sync_copy(z128.at[pl.ds(0, ZR)], acc.at[pl.ds(sid * ZR, ZR)])

      def flush(out0, out1):
          @pl.when(cid == 0)
          def _():
              pltpu.sync_copy(acc.at[pl.ds(sid * ZR, ZR)],
                              out0.at[pl.ds(sid * ZR, ZR)])

          @pl.when(cid == 1)
          def _():
              pltpu.sync_copy(acc.at[pl.ds(sid * ZR, ZR)],
                              out1.at[pl.ds(sid * ZR, ZR)])

      zero_stripe()

      pltpu.sync_copy(rel.at[wid], ridx)
      for c in range(IT_NCH):
          pltpu.async_copy(items_hbm.at[ridx.at[c]],
                           rows128.at[0, pl.ds(0, IT_CH)], sem_g)
          pltpu.make_async_copy(items_hbm.at[ridx.at[c]],
                                rows128.at[0, pl.ds(0, IT_CH)], sem_g).wait()
          pltpu.sync_copy(
              rows128.at[0, pl.ds(0, IT_CH)],
              out_items.at[pl.ds(wid * IT_NCH * IT_CH + c * IT_CH, IT_CH)])

      plsc.subcore_barrier()

      @pl.when(cid == 0)
      def _():
          _edge_loop(ops_hbm, acc, prec1, prec0, ebase, sidx, didx, rows128,
                     sem_g, sem_i)

      @pl.when(cid == 1)
      def _():
          _edge_loop(ops_hbm, acc, prec0, prec1, ebase, sidx, didx, rows128,
                     sem_g, sem_i)

      plsc.subcore_barrier()
      flush(out_preds, out_succs)
      plsc.subcore_barrier()
      zero_stripe()
      plsc.subcore_barrier()

      @pl.when(cid == 0)
      def _():
          _edge_loop(rm_hbm, acc, rese1, rese0, ebase, sidx, didx, rows128,
                     sem_g, sem_i)

      @pl.when(cid == 1)
      def _():
          _edge_loop(rm_hbm, acc, mate1, mate0, ebase, sidx, didx, rows128,
                     sem_g, sem_i)

      plsc.subcore_barrier()
      flush(out_res, out_mat)

  return _sc_gather_agg


def _elu(x):
    return jnp.where(x > 0, x, jnp.exp(x) - 1.0)


def _tc_body(ops_ref, itm_ref, agp_ref, ags_ref, agr_ref, agm_ref, *refs):
    prefs, out_ref = refs[:-1], refs[-1]

    def layers(i):
        base = i * 6
        return [(prefs[base + 2 * k], prefs[base + 2 * k + 1])
                for k in range(3)]

    def mlp(x, ws):
        for k, (w, b) in enumerate(ws):
            x = jnp.dot(x, w[...], preferred_element_type=jnp.float32) + b[...]
            if k < 2:
                x = _elu(x)
        return x

    p = mlp(agp_ref[...], layers(0))
    s = mlp(ags_ref[...], layers(1))
    r = mlp(agr_ref[:, :64], layers(2))
    m = mlp(agm_ref[:, 64:], layers(3))
    it = mlp(itm_ref[...], layers(4))
    se = mlp(ops_ref[...], layers(5))
    comb = jnp.concatenate([p, s, r, m, it, se], axis=-1)
    out_ref[...] = mlp(comb, layers(6))


_BROW = 400


def _tc_mlps(ops, itm, agp, ags, agr, agm, flat_params):
    row_in = [
        pl.BlockSpec((_BROW, 128), lambda i: (i, 0)),
        pl.BlockSpec((_BROW, 128), lambda i: (i, 0)),
        pl.BlockSpec((_BROW, 128), lambda i: (i, 0)),
        pl.BlockSpec((_BROW, 128), lambda i: (i, 0)),
        pl.BlockSpec((_BROW, 128), lambda i: (i, 0)),
        pl.BlockSpec((_BROW, 128), lambda i: (i, 0)),
    ]
    w_in = [pl.BlockSpec(w.shape, lambda i: (0,) * w.ndim) for w in flat_params]
    return pl.pallas_call(
        _tc_body,
        grid=(N // _BROW,),
        in_specs=row_in + w_in,
        out_specs=pl.BlockSpec((_BROW, 128), lambda i: (i, 0)),
        out_shape=jax.ShapeDtypeStruct((N, 128), jnp.float32),
        compiler_params=pltpu.CompilerParams(
            dimension_semantics=("arbitrary",)),
    )(ops, itm, agp, ags, agr, agm, *flat_params)


def kernel(operations, items, related_items, materials, resources,
           need_for_resources_edge_index, need_for_materials_edge_index,
           precedences_edge_index, params):
    i32 = jnp.int32
    prec = precedences_edge_index.astype(i32)
    rese = need_for_resources_edge_index.astype(i32)
    mate = need_for_materials_edge_index.astype(i32)

    rel = jnp.concatenate(
        [related_items.astype(i32),
         jnp.zeros((NI - N,), i32)]).reshape(32, IT_NCH, IT_CH)
    z128 = jnp.zeros((ZR, 128), jnp.float32)
    rm = jnp.concatenate([resources, materials], axis=1)

    it_rows, agp, ags, agr, agm = _get_sc_kernel()(
        operations, items, rm, prec[0], prec[1], rese[0], rese[1],
        mate[0], mate[1], rel, z128)

    flat_params = []
    for name in ("predecessors", "successors", "resources", "materials",
                 "items", "self", "combined"):
        for layer in params[name]:
            flat_params.append(layer["W"])
            flat_params.append(layer["b"].reshape(1, -1))

    return _tc_mlps(operations, it_rows, agp, ags, agr, agm, flat_params)
